# baseline (speedup 1.0000x reference)
import jax
import jax.numpy as jnp
from jax import lax
from jax.experimental import pallas as pl
from jax.experimental.pallas import tpu as pltpu
from jax.experimental.pallas import tpu_sc as plsc

XK = 1000000
NBATCH = 16384
SMOOTH_EPS = 1e-6

NC = 2
NS = 16
NW = NC * NS
BPW = NBATCH // NW
ROWS_PER_W = BPW // 128


MAIN_N = 999424
MAIN_ROWS = 7808
TAIL_N = XK - MAIN_N
STATS_GRID = 16
STATS_ROWS = MAIN_ROWS // STATS_GRID


def _stats_body(main_ref, tail_ref, out_ref, accm_ref, accs_ref):
    i = pl.program_id(0)
    x = main_ref[...]
    bm = jnp.max(x, axis=0, keepdims=True)

    @pl.when(i == 0)
    def _():
        accm_ref[...] = bm
        accs_ref[...] = jnp.sum(jnp.exp(x - bm), axis=0, keepdims=True)

    @pl.when(i > 0)
    def _():
        m_old = accm_ref[...]
        new_m = jnp.maximum(m_old, bm)
        accs_ref[...] = accs_ref[...] * jnp.exp(m_old - new_m) + jnp.sum(
            jnp.exp(x - new_m), axis=0, keepdims=True
        )
        accm_ref[...] = new_m

    @pl.when(i == pl.num_programs(0) - 1)
    def _():
        t = tail_ref[...]
        m_lane = accm_ref[...]
        m = jnp.maximum(jnp.max(m_lane), jnp.max(t))
        s = jnp.sum(accs_ref[...] * jnp.exp(m_lane - m)) + jnp.sum(jnp.exp(t - m))
        scale = (1.0 - SMOOTH_EPS) / s
        row = lax.broadcasted_iota(jnp.int32, (8, 128), 0)
        out_ref[...] = jnp.where(row < 1, m, scale)


def _gather_body(emb_hbm, idx_hbm, out_hbm, idx_v, rows_v, sem):
    wid = lax.axis_index("s") * NC + lax.axis_index("c")
    base = wid * ROWS_PER_W
    pltpu.sync_copy(idx_hbm.at[pl.ds(base, ROWS_PER_W)], idx_v)
    copies = [
        pltpu.async_copy(emb_hbm.at[idx_v.at[j]], rows_v.at[j], sem)
        for j in range(ROWS_PER_W)
    ]
    for c in copies:
        c.wait()
    pltpu.sync_copy(rows_v, out_hbm.at[pl.ds(base, ROWS_PER_W)])


def _apply_body(g_ref, stats_ref, out_ref):
    s = stats_ref[...]
    m = s[0, 0]
    scale = s[1, 0]
    g = g_ref[...]
    out_ref[...] = jnp.exp(g - m) * scale + jnp.float32(SMOOTH_EPS / XK)


@jax.jit
def kernel(inputs, embedding):
    idx = inputs.reshape(NBATCH // 128, 128).astype(jnp.int32)

    mesh = plsc.VectorSubcoreMesh(core_axis_name="c", subcore_axis_name="s")
    gathered = pl.kernel(
        _gather_body,
        mesh=mesh,
        out_type=jax.ShapeDtypeStruct((NBATCH // 128, 128), jnp.float32),
        scratch_types=[
            pltpu.VMEM((ROWS_PER_W, 128), jnp.int32),
            pltpu.VMEM((ROWS_PER_W, 128), jnp.float32),
            pltpu.SemaphoreType.DMA,
        ],
    )(embedding, idx)

    main2d = lax.slice(embedding, (0,), (MAIN_N,)).reshape(MAIN_ROWS, 128)
    tail = lax.slice(embedding, (MAIN_N,), (XK,))
    stats = pl.pallas_call(
        _stats_body,
        grid=(STATS_GRID,),
        in_specs=[
            pl.BlockSpec((STATS_ROWS, 128), lambda i: (i, 0)),
            pl.BlockSpec((TAIL_N,), lambda i: (0,)),
        ],
        out_specs=pl.BlockSpec((8, 128), lambda i: (0, 0)),
        out_shape=jax.ShapeDtypeStruct((8, 128), jnp.float32),
        scratch_shapes=[
            pltpu.VMEM((1, 128), jnp.float32),
            pltpu.VMEM((1, 128), jnp.float32),
        ],
    )(main2d, tail)

    out = pl.pallas_call(
        _apply_body,
        out_shape=jax.ShapeDtypeStruct((NBATCH // 128, 128), jnp.float32),
    )(gathered, stats)

    return out.reshape(NBATCH, 1)


# speedup vs baseline: 1.1302x; 1.1302x over previous
import jax
import jax.numpy as jnp
from jax import lax
from jax.experimental import pallas as pl
from jax.experimental.pallas import tpu as pltpu
from jax.experimental.pallas import tpu_sc as plsc

XK = 1000000
NBATCH = 16384
SMOOTH_EPS = 1e-6

NC = 2
NS = 16
NW = NC * NS
BPW = NBATCH // NW
ROWS_PER_W = BPW // 128


MAIN_N = 999424
TAIL_N = XK - MAIN_N
NCHUNK = 8
CHUNK_ROWS = 976
CHUNK_ELEMS = CHUNK_ROWS * 128


def _stats_body(main_hbm, out_ref, bufa, bufb, sema, semb):
    bufs = (bufa, bufb)
    sems = (sema, semb)

    def copy(k):
        src = main_hbm.at[pl.ds(k * CHUNK_ROWS, CHUNK_ROWS), :]
        return pltpu.make_async_copy(src, bufs[k % 2], sems[k % 2])

    copy(0).start()

    acc_m = None
    acc_s = None
    for k in range(NCHUNK):
        if k + 1 < NCHUNK:
            copy(k + 1).start()
        copy(k).wait()
        x = bufs[k % 2][...]
        bm = jnp.max(x, axis=0, keepdims=True)
        if acc_m is None:
            acc_m = bm
            acc_s = jnp.sum(jnp.exp(x - bm), axis=0, keepdims=True)
        else:
            new_m = jnp.maximum(acc_m, bm)
            acc_s = acc_s * jnp.exp(acc_m - new_m) + jnp.sum(
                jnp.exp(x - new_m), axis=0, keepdims=True
            )
            acc_m = new_m

    m = jnp.max(acc_m)
    s = jnp.sum(acc_s * jnp.exp(acc_m - m))
    row = lax.broadcasted_iota(jnp.int32, (8, 128), 0)
    out_ref[...] = jnp.where(row < 1, m, s)


def _gather_body(emb_hbm, idx_hbm, out_hbm, idx_v, rows_v, sem):
    wid = lax.axis_index("s") * NC + lax.axis_index("c")
    base = wid * ROWS_PER_W
    pltpu.sync_copy(idx_hbm.at[pl.ds(base, ROWS_PER_W)], idx_v)
    copies = [
        pltpu.async_copy(emb_hbm.at[idx_v.at[j]], rows_v.at[j], sem)
        for j in range(ROWS_PER_W)
    ]
    for c in copies:
        c.wait()
    pltpu.sync_copy(rows_v, out_hbm.at[pl.ds(base, ROWS_PER_W)])


def _apply_body(g_ref, stats_ref, emb_hbm, out_ref, tailb, semt):
    cp = pltpu.make_async_copy(emb_hbm.at[pl.ds(MAIN_N, TAIL_N)], tailb, semt)
    cp.start()
    st = stats_ref[...]
    m_main = st[0, 0]
    s_main = st[1, 0]
    cp.wait()
    t = tailb[...]
    m = jnp.maximum(m_main, jnp.max(t))
    s = s_main * jnp.exp(m_main - m) + jnp.sum(jnp.exp(t - m))
    scale = (1.0 - SMOOTH_EPS) / s
    g = g_ref[...]
    out_ref[...] = jnp.exp(g - m) * scale + jnp.float32(SMOOTH_EPS / XK)


@jax.jit
def kernel(inputs, embedding):
    idx = inputs.reshape(NBATCH // 128, 128).astype(jnp.int32)

    mesh = plsc.VectorSubcoreMesh(core_axis_name="c", subcore_axis_name="s")
    gathered = pl.kernel(
        _gather_body,
        mesh=mesh,
        out_type=jax.ShapeDtypeStruct((NBATCH // 128, 128), jnp.float32),
        scratch_types=[
            pltpu.VMEM((ROWS_PER_W, 128), jnp.int32),
            pltpu.VMEM((ROWS_PER_W, 128), jnp.float32),
            pltpu.SemaphoreType.DMA,
        ],
    )(embedding, idx)

    main2d = lax.slice(embedding, (0,), (MAIN_N,)).reshape(MAIN_N // 128, 128)
    stats = pl.pallas_call(
        _stats_body,
        in_specs=[pl.BlockSpec(memory_space=pl.ANY)],
        out_shape=jax.ShapeDtypeStruct((8, 128), jnp.float32),
        scratch_shapes=[
            pltpu.VMEM((CHUNK_ROWS, 128), jnp.float32),
            pltpu.VMEM((CHUNK_ROWS, 128), jnp.float32),
            pltpu.SemaphoreType.DMA,
            pltpu.SemaphoreType.DMA,
        ],
    )(main2d)

    out = pl.pallas_call(
        _apply_body,
        in_specs=[
            pl.BlockSpec(memory_space=pltpu.VMEM),
            pl.BlockSpec(memory_space=pltpu.VMEM),
            pl.BlockSpec(memory_space=pl.ANY),
        ],
        out_shape=jax.ShapeDtypeStruct((NBATCH // 128, 128), jnp.float32),
        scratch_shapes=[
            pltpu.VMEM((TAIL_N,), jnp.float32),
            pltpu.SemaphoreType.DMA,
        ],
    )(gathered, stats, embedding)

    return out.reshape(NBATCH, 1)


# speedup vs baseline: 1.1870x; 1.0502x over previous
import jax
import jax.numpy as jnp
from jax import lax
from jax.experimental import pallas as pl
from jax.experimental.pallas import tpu as pltpu
from jax.experimental.pallas import tpu_sc as plsc

XK = 1000000
NBATCH = 16384
SMOOTH_EPS = 1e-6

NC = 2
NS = 16
NW = NC * NS
BPW = NBATCH // NW
ROWS_PER_W = BPW // 128


PAD_ROWS = 7936
PAD_N = PAD_ROWS * 128
NCHUNK = 16
CHUNK_ROWS = PAD_ROWS // NCHUNK
NBUF = 4


def _stats_body(main_hbm, out_ref, bufa, bufb, bufc, bufd, sema, semb, semc, semd):
    bufs = (bufa, bufb, bufc, bufd)
    sems = (sema, semb, semc, semd)

    def copy(k):
        src = main_hbm.at[pl.ds(k * CHUNK_ROWS, CHUNK_ROWS), :]
        return pltpu.make_async_copy(src, bufs[k % NBUF], sems[k % NBUF])

    for k in range(NBUF):
        copy(k).start()

    acc_m = None
    acc_s = None
    for k in range(NCHUNK):
        copy(k).wait()
        x = bufs[k % NBUF][...]
        if k + NBUF < NCHUNK:
            copy(k + NBUF).start()
        bm = jnp.max(x, axis=0, keepdims=True)
        if acc_m is None:
            acc_m = bm
            acc_s = jnp.sum(jnp.exp(x - bm), axis=0, keepdims=True)
        else:
            new_m = jnp.maximum(acc_m, bm)
            acc_s = acc_s * jnp.exp(acc_m - new_m) + jnp.sum(
                jnp.exp(x - new_m), axis=0, keepdims=True
            )
            acc_m = new_m

    m = jnp.max(acc_m)
    s = jnp.sum(acc_s * jnp.exp(acc_m - m))
    scale = (1.0 - SMOOTH_EPS) / s
    row = lax.broadcasted_iota(jnp.int32, (8, 128), 0)
    out_ref[...] = jnp.where(row < 1, m, scale)


def _gather_body(emb_hbm, idx_hbm, out_hbm, idx_v, rows_v, sem):
    wid = lax.axis_index("s") * NC + lax.axis_index("c")
    base = wid * ROWS_PER_W
    pltpu.sync_copy(idx_hbm.at[pl.ds(base, ROWS_PER_W)], idx_v)
    copies = [
        pltpu.async_copy(emb_hbm.at[idx_v.at[j]], rows_v.at[j], sem)
        for j in range(ROWS_PER_W)
    ]
    for c in copies:
        c.wait()
    pltpu.sync_copy(rows_v, out_hbm.at[pl.ds(base, ROWS_PER_W)])


def _apply_body(g_ref, stats_ref, out_ref):
    st = stats_ref[...]
    m = st[0, 0]
    scale = st[1, 0]
    g = g_ref[...]
    out_ref[...] = jnp.exp(g - m) * scale + jnp.float32(SMOOTH_EPS / XK)


@jax.jit
def kernel(inputs, embedding):
    idx = inputs.reshape(NBATCH // 128, 128).astype(jnp.int32)

    mesh = plsc.VectorSubcoreMesh(core_axis_name="c", subcore_axis_name="s")
    gathered = pl.kernel(
        _gather_body,
        mesh=mesh,
        out_type=jax.ShapeDtypeStruct((NBATCH // 128, 128), jnp.float32),
        scratch_types=[
            pltpu.VMEM((ROWS_PER_W, 128), jnp.int32),
            pltpu.VMEM((ROWS_PER_W, 128), jnp.float32),
            pltpu.SemaphoreType.DMA,
        ],
    )(embedding, idx)

    emb_pad = jnp.pad(
        embedding, (0, PAD_N - XK), constant_values=-jnp.inf
    ).reshape(PAD_ROWS, 128)
    stats = pl.pallas_call(
        _stats_body,
        in_specs=[pl.BlockSpec(memory_space=pl.ANY)],
        out_shape=jax.ShapeDtypeStruct((8, 128), jnp.float32),
        scratch_shapes=[pltpu.VMEM((CHUNK_ROWS, 128), jnp.float32)] * NBUF
        + [pltpu.SemaphoreType.DMA] * NBUF,
    )(emb_pad)

    out = pl.pallas_call(
        _apply_body,
        out_shape=jax.ShapeDtypeStruct((NBATCH // 128, 128), jnp.float32),
    )(gathered, stats)

    return out.reshape(NBATCH, 1)


# speedup vs baseline: 1.2760x; 1.0750x over previous
import jax
import jax.numpy as jnp
from jax import lax
from jax.experimental import pallas as pl
from jax.experimental.pallas import tpu as pltpu
from jax.experimental.pallas import tpu_sc as plsc

XK = 1000000
NBATCH = 16384
SMOOTH_EPS = 1e-6

NC = 2
NS = 16
NW = NC * NS
BPW = NBATCH // NW
ROWS_PER_W = BPW // 128


PAD_ROWS = 7936
PAD_N = PAD_ROWS * 128
NCHUNK = 16
CHUNK_ROWS = PAD_ROWS // NCHUNK
NBUF = 4


def _stats_body(main_ref, out_ref):
    acc_m = None
    acc_s = None
    for k in range(NCHUNK):
        x = main_ref[pl.ds(k * CHUNK_ROWS, CHUNK_ROWS), :]
        bm = jnp.max(x, axis=0, keepdims=True)
        if acc_m is None:
            acc_m = bm
            acc_s = jnp.sum(jnp.exp(x - bm), axis=0, keepdims=True)
        else:
            new_m = jnp.maximum(acc_m, bm)
            acc_s = acc_s * jnp.exp(acc_m - new_m) + jnp.sum(
                jnp.exp(x - new_m), axis=0, keepdims=True
            )
            acc_m = new_m

    m = jnp.max(acc_m)
    s = jnp.sum(acc_s * jnp.exp(acc_m - m))
    scale = (1.0 - SMOOTH_EPS) / s
    row = lax.broadcasted_iota(jnp.int32, (8, 128), 0)
    out_ref[...] = jnp.where(row < 1, m, scale)


def _gather_body(emb_hbm, idx_hbm, out_hbm, idx_v, rows_v, sem):
    wid = lax.axis_index("s") * NC + lax.axis_index("c")
    base = wid * ROWS_PER_W
    pltpu.sync_copy(idx_hbm.at[pl.ds(base, ROWS_PER_W)], idx_v)
    copies = [
        pltpu.async_copy(emb_hbm.at[idx_v.at[j]], rows_v.at[j], sem)
        for j in range(ROWS_PER_W)
    ]
    for c in copies:
        c.wait()
    pltpu.sync_copy(rows_v, out_hbm.at[pl.ds(base, ROWS_PER_W)])


def _apply_body(g_ref, stats_ref, out_ref):
    st = stats_ref[...]
    m = st[0, 0]
    scale = st[1, 0]
    g = g_ref[...]
    out_ref[...] = jnp.exp(g - m) * scale + jnp.float32(SMOOTH_EPS / XK)


@jax.jit
def kernel(inputs, embedding):
    idx = inputs.reshape(NBATCH // 128, 128).astype(jnp.int32)

    mesh = plsc.VectorSubcoreMesh(core_axis_name="c", subcore_axis_name="s")
    gathered = pl.kernel(
        _gather_body,
        mesh=mesh,
        out_type=jax.ShapeDtypeStruct((NBATCH // 128, 128), jnp.float32),
        scratch_types=[
            pltpu.VMEM((ROWS_PER_W, 128), jnp.int32),
            pltpu.VMEM((ROWS_PER_W, 128), jnp.float32),
            pltpu.SemaphoreType.DMA,
        ],
    )(embedding, idx)

    emb_pad = jnp.concatenate(
        [embedding, jnp.full((PAD_N - XK,), -jnp.inf, jnp.float32)]
    ).reshape(PAD_ROWS, 128)
    stats = pl.pallas_call(
        _stats_body,
        out_shape=jax.ShapeDtypeStruct((8, 128), jnp.float32),
    )(emb_pad)

    out = pl.pallas_call(
        _apply_body,
        out_shape=jax.ShapeDtypeStruct((NBATCH // 128, 128), jnp.float32),
    )(gathered, stats)

    return out.reshape(NBATCH, 1)
